# baseline (speedup 1.0000x reference)
import functools

import jax
import jax.numpy as jnp
from jax import lax
from jax.experimental import pallas as pl
from jax.experimental.pallas import tpu as pltpu
from jax.experimental.pallas import tpu_sc as plsc

N = 10000
E = 160000
F = 32
H = 8
C = 32
G = 16
NB = 2000
NBLK = N // NB
EP = 172032
EV = E + N


def _k1_body(x_ref, w_ref, as_ref, ad_ref, h1a_ref, h1b_ref, s_ref, d_ref):
    h1 = jnp.dot(x_ref[...], w_ref[...], preferred_element_type=jnp.float32, precision=lax.Precision.HIGHEST)
    h1a_ref[...] = h1[:, :128]
    h1b_ref[...] = h1[:, 128:]
    s_ref[...] = jnp.dot(h1, as_ref[...], preferred_element_type=jnp.float32, precision=lax.Precision.HIGHEST)
    d_ref[...] = jnp.dot(h1, ad_ref[...], preferred_element_type=jnp.float32, precision=lax.Precision.HIGHEST)


def _k1(x, w_gat, asads, asadd):
    return pl.pallas_call(
        _k1_body,
        grid=(NBLK,),
        in_specs=[
            pl.BlockSpec((NB, F), lambda i: (i, 0)),
            pl.BlockSpec((F, H * C), lambda i: (0, 0)),
            pl.BlockSpec((H * C, 16), lambda i: (0, 0)),
            pl.BlockSpec((H * C, 16), lambda i: (0, 0)),
        ],
        out_specs=[
            pl.BlockSpec((NB, 128), lambda i: (i, 0)),
            pl.BlockSpec((NB, 128), lambda i: (i, 0)),
            pl.BlockSpec((NB, 16), lambda i: (i, 0)),
            pl.BlockSpec((NB, 16), lambda i: (i, 0)),
        ],
        out_shape=[
            jax.ShapeDtypeStruct((N, 128), jnp.float32),
            jax.ShapeDtypeStruct((N, 128), jnp.float32),
            jax.ShapeDtypeStruct((N, 16), jnp.float32),
            jax.ShapeDtypeStruct((N, 16), jnp.float32),
        ],
    )(x, w_gat, asads, asadd)


def _dinv_from_dd(dd):
    deg = dd[:, 8:9]
    return jnp.where(deg > 0, lax.rsqrt(jnp.maximum(deg, 1e-12)), 0.0)


def _k3_body(ga_ref, gb_ref, dd_ref, bg_ref, w2_ref, out_ref):
    hg = jnp.concatenate([ga_ref[...], gb_ref[...]], axis=1) + bg_ref[...]
    hg = jnp.maximum(hg, 0.0)
    dinv = _dinv_from_dd(dd_ref[...])
    out_ref[0] = dinv * jnp.dot(hg, w2_ref[...],
                                preferred_element_type=jnp.float32, precision=lax.Precision.HIGHEST)


def _k3(gata, gatb, dd, b_gat, w2):
    return pl.pallas_call(
        _k3_body,
        grid=(NBLK, 4),
        in_specs=[
            pl.BlockSpec((NB, 128), lambda i, c: (i, 0)),
            pl.BlockSpec((NB, 128), lambda i, c: (i, 0)),
            pl.BlockSpec((NB, 16), lambda i, c: (i, 0)),
            pl.BlockSpec((1, H * C), lambda i, c: (0, 0)),
            pl.BlockSpec((H * C, 128), lambda i, c: (0, c)),
        ],
        out_specs=pl.BlockSpec((1, NB, 128), lambda i, c: (c, i, 0)),
        out_shape=jax.ShapeDtypeStruct((4, N, 128), jnp.float32),
    )(gata, gatb, dd, b_gat, w2)


def _k5_body(nchunk, a_refs_dd_b_t_s):
    refs = a_refs_dd_b_t_s
    a_refs = refs[:nchunk]
    dd_ref, b_ref, t_ref, s_ref = refs[nchunk:]
    i = pl.program_id(0)
    t = jnp.concatenate([r[0] for r in a_refs], axis=1)
    dinv = _dinv_from_dd(dd_ref[...])
    t = dinv * t + b_ref[...]
    t_ref[...] = t
    s = jnp.concatenate([jnp.sum(t, axis=0, keepdims=True),
                         jnp.sum(t * t, axis=0, keepdims=True)], axis=0)
    prev = jnp.where(i == 0, jnp.zeros_like(s), s_ref[...])
    s_ref[...] = prev + s


def _k5(acc, dd, bias, width, nchunk):
    def body(*refs):
        _k5_body(nchunk, refs)
    in_specs = [pl.BlockSpec((1, NB, 128), functools.partial(
        lambda c, i: (c, i, 0), c)) for c in range(nchunk)]
    in_specs += [
        pl.BlockSpec((NB, 16), lambda i: (i, 0)),
        pl.BlockSpec((1, width), lambda i: (0, 0)),
    ]
    return pl.pallas_call(
        body,
        grid=(NBLK,),
        in_specs=in_specs,
        out_specs=[
            pl.BlockSpec((NB, width), lambda i: (i, 0)),
            pl.BlockSpec((2, width), lambda i: (0, 0)),
        ],
        out_shape=[
            jax.ShapeDtypeStruct((N, width), jnp.float32),
            jax.ShapeDtypeStruct((2, width), jnp.float32),
        ],
    )(*([acc] * nchunk), dd, bias)


def _k6_body(t_ref, s_ref, g_ref, be_ref, w3_ref, dd_ref, out_ref):
    s = s_ref[...]
    mu = s[0:1] / N
    var = s[1:2] / N - mu * mu
    h3 = (t_ref[...] - mu) * lax.rsqrt(var + 1e-5) * g_ref[...] + be_ref[...]
    h3 = jnp.maximum(h3, 0.0)
    dinv = _dinv_from_dd(dd_ref[...])
    out_ref[0] = dinv * jnp.dot(h3, w3_ref[...],
                                preferred_element_type=jnp.float32, precision=lax.Precision.HIGHEST)


def _k6(t, stats, g1, be1, w3, dd):
    return pl.pallas_call(
        _k6_body,
        grid=(NBLK, 8),
        in_specs=[
            pl.BlockSpec((NB, 512), lambda i, c: (i, 0)),
            pl.BlockSpec((2, 512), lambda i, c: (0, 0)),
            pl.BlockSpec((1, 512), lambda i, c: (0, 0)),
            pl.BlockSpec((1, 512), lambda i, c: (0, 0)),
            pl.BlockSpec((512, 128), lambda i, c: (0, c)),
            pl.BlockSpec((NB, 16), lambda i, c: (i, 0)),
        ],
        out_specs=pl.BlockSpec((1, NB, 128), lambda i, c: (c, i, 0)),
        out_shape=jax.ShapeDtypeStruct((8, N, 128), jnp.float32),
    )(t, stats, g1, be1, w3, dd)


def _k7b_body(s_ref, g_ref, be_ref, ab_ref):
    s = s_ref[...]
    mu = s[0:1] / N
    var = s[1:2] / N - mu * mu
    a = g_ref[...] * lax.rsqrt(var + 1e-5)
    b = be_ref[...] - mu * a
    ab_ref[...] = jnp.concatenate([a, b], axis=0)


def _k7b(stats, g2, be2):
    return pl.pallas_call(
        _k7b_body,
        grid=(1,),
        in_specs=[
            pl.BlockSpec((2, 1024), lambda i: (0, 0)),
            pl.BlockSpec((1, 1024), lambda i: (0, 0)),
            pl.BlockSpec((1, 1024), lambda i: (0, 0)),
        ],
        out_specs=pl.BlockSpec((2, 1024), lambda i: (0, 0)),
        out_shape=jax.ShapeDtypeStruct((2, 1024), jnp.float32),
    )(stats, g2, be2)


def _k8_body(p_ref, w1_ref, b1_ref, w2_ref, b2_ref, out_ref):
    z = jnp.dot(p_ref[...], w1_ref[...], preferred_element_type=jnp.float32, precision=lax.Precision.HIGHEST)
    z = jnp.maximum(z + b1_ref[...], 0.0)
    out_ref[...] = jnp.dot(z, w2_ref[...],
                           preferred_element_type=jnp.float32, precision=lax.Precision.HIGHEST) + b2_ref[...]


def _k8(pooled, w_fc1, b_fc1, w_fc2p, b_fc2p):
    return pl.pallas_call(
        _k8_body,
        grid=(1,),
        in_specs=[
            pl.BlockSpec((G, 2048), lambda i: (0, 0)),
            pl.BlockSpec((2048, 256), lambda i: (0, 0)),
            pl.BlockSpec((1, 256), lambda i: (0, 0)),
            pl.BlockSpec((256, 128), lambda i: (0, 0)),
            pl.BlockSpec((1, 128), lambda i: (0, 0)),
        ],
        out_specs=pl.BlockSpec((G, 128), lambda i: (0, 0)),
        out_shape=jax.ShapeDtypeStruct((G, 128), jnp.float32),
    )(pooled, w_fc1, b_fc1, w_fc2p, b_fc2p)


EPT = EP // 16
NBATCH = EPT // 128
RPT = N // 16


def _sc_mesh():
    return plsc.VectorSubcoreMesh(core_axis_name="c", subcore_axis_name="s",
                                  num_cores=2, num_subcores=16)


def _lane():
    return jnp.arange(16, dtype=jnp.int32)


def _zero_zbuf(zbuf, width):
    nj = width // 16

    def zb(i, c):
        for j in range(nj):
            zbuf[i, pl.ds(16 * j, 16)] = jnp.zeros((16,), jnp.float32)
        return c
    lax.fori_loop(0, zbuf.shape[0], zb, None)


def _zero_acc(zbuf, acc, sid):
    r0 = sid * RPT
    rows = zbuf.shape[0]
    for k in range(RPT // rows):
        pltpu.sync_copy(zbuf, acc.at[pl.ds(r0 + k * rows, rows)])


def _sc_b1_body(arows, arowd, srcp, dstp, wp, dd_out, exw_out,
                idx_s, idx_d, sbuf, dbuf, robuf, wbuf, zbuf, sem, acc):
    cid = lax.axis_index("c")
    sid = lax.axis_index("s")
    lmask = _lane() < 8

    @pl.when(cid == 0)
    def _():
        def zb(i, c):
            zbuf[i] = jnp.zeros((16,), jnp.float32)
            return c
        lax.fori_loop(0, 125, zb, None)
        _zero_acc(zbuf, acc, sid)
        plsc.subcore_barrier()
        ebase = sid * EPT

        def batch_body(b, c):
            base = ebase + b * 128
            pltpu.sync_copy(srcp.at[pl.ds(base, 128)], idx_s)
            pltpu.sync_copy(dstp.at[pl.ds(base, 128)], idx_d)
            pltpu.sync_copy(wp.at[pl.ds(base, 128)], wbuf)
            pltpu.async_copy(arows.at[idx_s], sbuf, sem).wait()
            pltpu.async_copy(arowd.at[idx_d], dbuf, sem).wait()

            def edge_body(i, c2):
                a = sbuf[i] + dbuf[i]
                a = jnp.maximum(a, 0.2 * a)
                ex = jnp.exp(a)
                valid = (base + i) < EV
                robuf[i] = jnp.where(jnp.logical_and(lmask, valid), ex, 0.0)
                return c2
            lax.fori_loop(0, 128, edge_body, None)
            for g in range(8):
                plsc.store_scatter(
                    robuf,
                    [_lane() + 16 * g, jnp.full((16,), 8, jnp.int32)],
                    wbuf[pl.ds(16 * g, 16)])
            pltpu.sync_copy(robuf, exw_out.at[pl.ds(base, 128)])
            pltpu.sync_copy(robuf, acc.at[idx_d], add=True)
            return c
        lax.fori_loop(0, NBATCH, batch_body, None)
        plsc.subcore_barrier()
        r0 = sid * RPT
        pltpu.sync_copy(acc.at[pl.ds(r0, RPT)], dd_out.at[pl.ds(r0, RPT)])


def _sc_b1(arows, arowd, srcp, dstp, wp):
    f = pl.kernel(
        _sc_b1_body,
        out_type=[jax.ShapeDtypeStruct((N, 16), jnp.float32),
                  jax.ShapeDtypeStruct((EP, 16), jnp.float32)],
        mesh=_sc_mesh(),
        compiler_params=pltpu.CompilerParams(use_tc_tiling_on_sc=False, needs_layout_passes=False),
        scratch_types=[
            pltpu.VMEM((128,), jnp.int32),
            pltpu.VMEM((128,), jnp.int32),
            pltpu.VMEM((128, 16), jnp.float32),
            pltpu.VMEM((128, 16), jnp.float32),
            pltpu.VMEM((128, 16), jnp.float32),
            pltpu.VMEM((128,), jnp.float32),
            pltpu.VMEM((125, 16), jnp.float32),
            pltpu.SemaphoreType.DMA,
            pltpu.VMEM_SHARED((N, 16), jnp.float32),
        ],
    )
    return f(arows, arowd, srcp, dstp, wp)


def _sc_b2_body(h1a, h1b, exw, dd, srcp, dstp, gata, gatb,
                idxsa, idxsb, idxda, idxdb, hbufa, hbufb, ddbufa, ddbufb,
                exbufa, exbufb, cfbuf, zbuf, gsema, gsemb, acc):
    cid = lax.axis_index("c")
    sid = lax.axis_index("s")
    idxs = (idxsa, idxsb)
    idxd = (idxda, idxdb)
    hbuf = (hbufa, hbufb)
    ddbuf = (ddbufa, ddbufb)
    exbuf = (exbufa, exbufb)
    gsem = (gsema, gsemb)

    def process(href, out_ref, headbase):
        _zero_zbuf(zbuf, 128)
        _zero_acc(zbuf, acc, sid)
        plsc.subcore_barrier()
        ebase = sid * EPT

        def prefetch(b, slot):
            base = ebase + b * 128
            pltpu.sync_copy(srcp.at[pl.ds(base, 128)], idxs[slot])
            pltpu.sync_copy(dstp.at[pl.ds(base, 128)], idxd[slot])
            pltpu.sync_copy(exw.at[pl.ds(base, 128)], exbuf[slot])
            pltpu.async_copy(href.at[idxs[slot]], hbuf[slot], gsem[slot])
            pltpu.async_copy(dd.at[idxd[slot]], ddbuf[slot], gsem[slot])

        def gwait(slot):
            pltpu.make_async_copy(
                href.at[idxs[slot]], hbuf[slot], gsem[slot]).wait()
            pltpu.make_async_copy(
                dd.at[idxd[slot]], ddbuf[slot], gsem[slot]).wait()

        prefetch(0, 0)

        def pair_body(k, c):
            for par in (0, 1):
                b = 2 * k + par
                nxt = 1 - par
                if par == 0:
                    prefetch(b + 1, nxt)
                else:
                    @pl.when(k < NBATCH // 2 - 1)
                    def _():
                        prefetch(b + 1, nxt)
                gwait(par)

                def edge_body(i, c2):
                    for u in range(2):
                        e = 2 * i + u
                        cfbuf[e] = exbuf[par][e] / ddbuf[par][e]
                        iv = jnp.zeros((16,), jnp.int32) + e
                        for hd in range(4):
                            cs = plsc.load_gather(
                                cfbuf,
                                [iv, jnp.full((16,), headbase + hd,
                                              jnp.int32)])
                            for j in (2 * hd, 2 * hd + 1):
                                hbuf[par][e, pl.ds(16 * j, 16)] = (
                                    hbuf[par][e, pl.ds(16 * j, 16)] * cs)
                    return c2
                lax.fori_loop(0, 64, edge_body, None)
                pltpu.sync_copy(hbuf[par], acc.at[idxd[par]], add=True)
            return c
        lax.fori_loop(0, NBATCH // 2, pair_body, None)
        plsc.subcore_barrier()
        r0 = sid * RPT
        pltpu.sync_copy(acc.at[pl.ds(r0, RPT)], out_ref.at[pl.ds(r0, RPT)])

    pl.when(cid == 0)(lambda: process(h1a, gata, 0))
    pl.when(cid == 1)(lambda: process(h1b, gatb, 4))


def _sc_b2(h1a, h1b, exw, dd, srcp, dstp):
    f = pl.kernel(
        _sc_b2_body,
        out_type=[jax.ShapeDtypeStruct((N, 128), jnp.float32),
                  jax.ShapeDtypeStruct((N, 128), jnp.float32)],
        mesh=_sc_mesh(),
        compiler_params=pltpu.CompilerParams(use_tc_tiling_on_sc=False, needs_layout_passes=False),
        scratch_types=[
            pltpu.VMEM((128,), jnp.int32),
            pltpu.VMEM((128,), jnp.int32),
            pltpu.VMEM((128,), jnp.int32),
            pltpu.VMEM((128,), jnp.int32),
            pltpu.VMEM((128, 128), jnp.float32),
            pltpu.VMEM((128, 128), jnp.float32),
            pltpu.VMEM((128, 16), jnp.float32),
            pltpu.VMEM((128, 16), jnp.float32),
            pltpu.VMEM((128, 16), jnp.float32),
            pltpu.VMEM((128, 16), jnp.float32),
            pltpu.VMEM((128, 16), jnp.float32),
            pltpu.VMEM((25, 128), jnp.float32),
            pltpu.SemaphoreType.DMA,
            pltpu.SemaphoreType.DMA,
            pltpu.VMEM_SHARED((N, 128), jnp.float32),
        ],
    )
    return f(h1a, h1b, exw, dd, srcp, dstp)


BT = 112
NBG = EPT // BT


def _sc_gcn_body(nc_per_sc, hs_flat, srcp, dstp, wp, out_flat,
                 idx_s, idx2a, idx2b, idx2c, idxda, idxdb, idxdc,
                 hbufa, hbufb, hbufc, wbufa, wbufb, wbufc, zbuf,
                 gsema, gsemb, gsemc, ssema, ssemb, ssemc, acc):
    cid = lax.axis_index("c")
    sid = lax.axis_index("s")
    idx2 = (idx2a, idx2b, idx2c)
    idxd = (idxda, idxdb, idxdc)
    hbuf = (hbufa, hbufb, hbufc)
    wbuf = (wbufa, wbufb, wbufc)
    gsem = (gsema, gsemb, gsemc)
    ssem = (ssema, ssemb, ssemc)
    _zero_zbuf(zbuf, 128)
    ebase = sid * EPT
    r0 = sid * RPT
    for cl in range(nc_per_sc):
        cg = cid * nc_per_sc + cl
        off = cg * N
        _zero_acc(zbuf, acc, sid)
        plsc.subcore_barrier()

        def prefetch(b, slot):
            base = ebase + b * BT
            pltpu.sync_copy(srcp.at[pl.ds(base, BT)], idx_s)
            pltpu.sync_copy(dstp.at[pl.ds(base, BT)], idxd[slot])
            pltpu.sync_copy(wp.at[pl.ds(base, BT)], wbuf[slot])
            for g in range(BT // 16):
                idx2[slot][pl.ds(16 * g, 16)] = (
                    idx_s[pl.ds(16 * g, 16)] + off)
            pltpu.async_copy(hs_flat.at[idx2[slot]], hbuf[slot], gsem[slot])

        def gwait(slot):
            pltpu.make_async_copy(
                hs_flat.at[idx2[slot]], hbuf[slot], gsem[slot]).wait()

        def swait(slot):
            pltpu.make_async_copy(
                hbuf[slot], acc.at[idxd[slot]], ssem[slot]).wait()

        prefetch(0, 0)
        prefetch(1, 1)

        def triple_body(k, c):
            for j in range(3):
                b = 3 * k + j
                p = (j + 2) % 3
                if j == 0:
                    @pl.when(k > 0)
                    def _():
                        swait(p)
                    prefetch(b + 2, p)
                else:
                    @pl.when(k < NBG // 3 - 1)
                    def _():
                        swait(p)
                        prefetch(b + 2, p)
                gwait(j)

                def edge_body(i, c2):
                    for u in range(2):
                        e = 2 * i + u
                        iv = jnp.zeros((16,), jnp.int32) + e
                        ws = plsc.load_gather(wbuf[j], [iv])
                        for q in range(8):
                            hbuf[j][e, pl.ds(16 * q, 16)] = (
                                hbuf[j][e, pl.ds(16 * q, 16)] * ws)
                    return c2
                lax.fori_loop(0, BT // 2, edge_body, None)
                pltpu.async_copy(hbuf[j], acc.at[idxd[j]], ssem[j],
                                 add=True)
            return c
        lax.fori_loop(0, NBG // 3, triple_body, None)
        swait(0)
        swait(1)
        swait(2)
        plsc.subcore_barrier()
        pltpu.sync_copy(acc.at[pl.ds(r0, RPT)],
                        out_flat.at[pl.ds(off + r0, RPT)])
        plsc.subcore_barrier()


def _sc_gcn(hchunks, srcp, dstp, wp):
    nc = hchunks.shape[0]
    hs_flat = hchunks.reshape(nc * N, 128)
    f = pl.kernel(
        functools.partial(_sc_gcn_body, nc // 2),
        out_type=jax.ShapeDtypeStruct((nc * N, 128), jnp.float32),
        mesh=_sc_mesh(),
        compiler_params=pltpu.CompilerParams(use_tc_tiling_on_sc=False, needs_layout_passes=False),
        scratch_types=[
            pltpu.VMEM((BT,), jnp.int32),
            pltpu.VMEM((BT,), jnp.int32),
            pltpu.VMEM((BT,), jnp.int32),
            pltpu.VMEM((BT,), jnp.int32),
            pltpu.VMEM((BT,), jnp.int32),
            pltpu.VMEM((BT,), jnp.int32),
            pltpu.VMEM((BT,), jnp.int32),
            pltpu.VMEM((BT, 128), jnp.float32),
            pltpu.VMEM((BT, 128), jnp.float32),
            pltpu.VMEM((BT, 128), jnp.float32),
            pltpu.VMEM((BT,), jnp.float32),
            pltpu.VMEM((BT,), jnp.float32),
            pltpu.VMEM((BT,), jnp.float32),
            pltpu.VMEM((25, 128), jnp.float32),
            pltpu.SemaphoreType.DMA,
            pltpu.SemaphoreType.DMA,
            pltpu.SemaphoreType.DMA,
            pltpu.SemaphoreType.DMA,
            pltpu.SemaphoreType.DMA,
            pltpu.SemaphoreType.DMA,
            pltpu.VMEM_SHARED((N, 128), jnp.float32),
        ],
    )
    return f(hs_flat, srcp, dstp, wp).reshape(nc, N, 128)


def _sc_pool_body(u, ab, rp, pooled,
                  ubuf, abbuf, obmax, obmean, rp_s, sem):
    cid = lax.axis_index("c")
    sid = lax.axis_index("s")
    wid = cid * 16 + sid
    c0 = wid * 32
    pltpu.sync_copy(rp, rp_s)
    pltpu.sync_copy(ab.at[:, pl.ds(c0, 32)], abbuf)
    a0 = abbuf[0, pl.ds(0, 16)]
    a1 = abbuf[0, pl.ds(16, 16)]
    b0 = abbuf[1, pl.ds(0, 16)]
    b1 = abbuf[1, pl.ds(16, 16)]
    rpv0 = rp_s[pl.ds(0, 16)]
    rpv1 = rp_s[pl.ds(16, 16)]
    for g in range(G):
        lo = rpv0[g]
        hi = rpv0[g + 1] if g + 1 < 16 else rpv1[0]
        cnt = hi - lo
        nblk = lax.div(cnt + 511, 512)

        def blk(bi, carry):
            blo = lo + bi * 512
            s = jnp.minimum(blo, N - 512)
            pltpu.sync_copy(u.at[pl.ds(s, 512), pl.ds(c0, 32)], ubuf)

            def row(i, c2):
                m0, m1, s0, s1 = c2
                r = s + i
                valid = jnp.logical_and(r >= blo, r < hi)
                v0 = jnp.maximum(ubuf[i, pl.ds(0, 16)] * a0 + b0, 0.0)
                v1 = jnp.maximum(ubuf[i, pl.ds(16, 16)] * a1 + b1, 0.0)
                v0 = jnp.where(valid, v0, 0.0)
                v1 = jnp.where(valid, v1, 0.0)
                return (jnp.maximum(m0, v0), jnp.maximum(m1, v1),
                        s0 + v0, s1 + v1)
            return lax.fori_loop(0, 512, row, carry)
        z = jnp.zeros((16,), jnp.float32)
        m0, m1, s0, s1 = lax.fori_loop(0, nblk, blk, (z, z, z, z))
        cntf = jnp.maximum(cnt.astype(jnp.float32), 1.0)
        obmax[g, pl.ds(0, 16)] = m0
        obmax[g, pl.ds(16, 16)] = m1
        obmean[g, pl.ds(0, 16)] = s0 / cntf
        obmean[g, pl.ds(16, 16)] = s1 / cntf
    pltpu.sync_copy(obmax, pooled.at[:, pl.ds(c0, 32)])
    pltpu.sync_copy(obmean, pooled.at[:, pl.ds(1024 + c0, 32)])


def _sc_pool(u, ab, rp32):
    f = pl.kernel(
        _sc_pool_body,
        out_type=jax.ShapeDtypeStruct((G, 2048), jnp.float32),
        mesh=_sc_mesh(),
        compiler_params=pltpu.CompilerParams(use_tc_tiling_on_sc=False, needs_layout_passes=False),
        scratch_types=[
            pltpu.VMEM((512, 32), jnp.float32),
            pltpu.VMEM((2, 32), jnp.float32),
            pltpu.VMEM((G, 32), jnp.float32),
            pltpu.VMEM((G, 32), jnp.float32),
            pltpu.VMEM((32,), jnp.int32),
            pltpu.SemaphoreType.DMA,
        ],
    )
    return f(u, ab, rp32)


def kernel(x, edge_index, edge_weights, batch, W_gat, att_src, att_dst,
           b_gat, W2, b2, W3, b3, g1, be1, g2, be2, W_fc1, b_fc1,
           W_fc2, b_fc2):
    loop = jnp.arange(N, dtype=edge_index.dtype)
    pad = jnp.zeros((EP - EV,), jnp.int32)
    srcp = jnp.concatenate([edge_index[0], loop, pad])
    dstp = jnp.concatenate([edge_index[1], loop, pad])
    wp = jnp.concatenate([edge_weights, jnp.ones((N,), jnp.float32),
                          jnp.zeros((EP - EV,), jnp.float32)])
    eye = jnp.eye(H, dtype=jnp.float32)
    zpad = jnp.zeros((H * C, 8), jnp.float32)
    asads = jnp.concatenate(
        [(eye[:, None, :] * att_src[:, :, None]).reshape(H * C, H), zpad], 1)
    asadd = jnp.concatenate(
        [(eye[:, None, :] * att_dst[:, :, None]).reshape(H * C, H), zpad], 1)
    rp = jnp.searchsorted(batch, jnp.arange(G + 1, dtype=jnp.int32)
                          ).astype(jnp.int32)
    rp32 = jnp.concatenate([rp, jnp.full((15,), N, jnp.int32)])
    w_fc2p = jnp.concatenate([W_fc2, jnp.zeros((256, 126), jnp.float32)], 1)
    b_fc2p = jnp.concatenate([b_fc2, jnp.zeros((126,), jnp.float32)])

    h1a, h1b, arows, arowd = _k1(x, W_gat, asads, asadd)
    dd, exw = _sc_b1(arows, arowd, srcp, dstp, wp)
    gata, gatb = _sc_b2(h1a, h1b, exw, dd, srcp, dstp)
    h2s = _k3(gata, gatb, dd, b_gat.reshape(1, -1), W2)
    g1acc = _sc_gcn(h2s, srcp, dstp, wp)
    t, stats1 = _k5(g1acc, dd, b2.reshape(1, -1), 512, 4)
    h3s = _k6(t, stats1, g1.reshape(1, -1), be1.reshape(1, -1), W3, dd)
    g2acc = _sc_gcn(h3s, srcp, dstp, wp)
    u, stats2 = _k5(g2acc, dd, b3.reshape(1, -1), 1024, 8)
    ab = _k7b(stats2, g2.reshape(1, -1), be2.reshape(1, -1))
    pooled = _sc_pool(u, ab, rp32)
    z = _k8(pooled, W_fc1, b_fc1.reshape(1, -1), w_fc2p,
            b_fc2p.reshape(1, -1))
    return z[:, :2]


# speedup vs baseline: 1.0135x; 1.0135x over previous
import functools

import jax
import jax.numpy as jnp
from jax import lax
from jax.experimental import pallas as pl
from jax.experimental.pallas import tpu as pltpu
from jax.experimental.pallas import tpu_sc as plsc

N = 10000
E = 160000
F = 32
H = 8
C = 32
G = 16
NB = 2000
NBLK = N // NB
EP = 172032
EV = E + N


def _k1_body(x_ref, w_ref, as_ref, ad_ref, h1a_ref, h1b_ref, s_ref, d_ref):
    h1 = jnp.dot(x_ref[...], w_ref[...], preferred_element_type=jnp.float32, precision=lax.Precision.HIGHEST)
    h1a_ref[...] = h1[:, :128]
    h1b_ref[...] = h1[:, 128:]
    s_ref[...] = jnp.dot(h1, as_ref[...], preferred_element_type=jnp.float32, precision=lax.Precision.HIGHEST)
    d_ref[...] = jnp.dot(h1, ad_ref[...], preferred_element_type=jnp.float32, precision=lax.Precision.HIGHEST)


def _k1(x, w_gat, asads, asadd):
    return pl.pallas_call(
        _k1_body,
        grid=(NBLK,),
        in_specs=[
            pl.BlockSpec((NB, F), lambda i: (i, 0)),
            pl.BlockSpec((F, H * C), lambda i: (0, 0)),
            pl.BlockSpec((H * C, 16), lambda i: (0, 0)),
            pl.BlockSpec((H * C, 16), lambda i: (0, 0)),
        ],
        out_specs=[
            pl.BlockSpec((NB, 128), lambda i: (i, 0)),
            pl.BlockSpec((NB, 128), lambda i: (i, 0)),
            pl.BlockSpec((NB, 16), lambda i: (i, 0)),
            pl.BlockSpec((NB, 16), lambda i: (i, 0)),
        ],
        out_shape=[
            jax.ShapeDtypeStruct((N, 128), jnp.float32),
            jax.ShapeDtypeStruct((N, 128), jnp.float32),
            jax.ShapeDtypeStruct((N, 16), jnp.float32),
            jax.ShapeDtypeStruct((N, 16), jnp.float32),
        ],
    )(x, w_gat, asads, asadd)


def _dinv_from_dd(dd):
    deg = dd[:, 8:9]
    return jnp.where(deg > 0, lax.rsqrt(jnp.maximum(deg, 1e-12)), 0.0)


def _k3_body(ga_ref, gb_ref, dd_ref, bg_ref, w2_ref, out_ref):
    hg = jnp.concatenate([ga_ref[...], gb_ref[...]], axis=1) + bg_ref[...]
    hg = jnp.maximum(hg, 0.0)
    dinv = _dinv_from_dd(dd_ref[...])
    out_ref[0] = dinv * jnp.dot(hg, w2_ref[...],
                                preferred_element_type=jnp.float32, precision=lax.Precision.HIGHEST)


def _k3(gata, gatb, dd, b_gat, w2):
    return pl.pallas_call(
        _k3_body,
        grid=(NBLK, 4),
        in_specs=[
            pl.BlockSpec((NB, 128), lambda i, c: (i, 0)),
            pl.BlockSpec((NB, 128), lambda i, c: (i, 0)),
            pl.BlockSpec((NB, 16), lambda i, c: (i, 0)),
            pl.BlockSpec((1, H * C), lambda i, c: (0, 0)),
            pl.BlockSpec((H * C, 128), lambda i, c: (0, c)),
        ],
        out_specs=pl.BlockSpec((1, NB, 128), lambda i, c: (c, i, 0)),
        out_shape=jax.ShapeDtypeStruct((4, N, 128), jnp.float32),
    )(gata, gatb, dd, b_gat, w2)


def _k5_body(nchunk, a_refs_dd_b_t_s):
    refs = a_refs_dd_b_t_s
    a_refs = refs[:nchunk]
    dd_ref, b_ref, t_ref, s_ref = refs[nchunk:]
    i = pl.program_id(0)
    t = jnp.concatenate([r[0] for r in a_refs], axis=1)
    dinv = _dinv_from_dd(dd_ref[...])
    t = dinv * t + b_ref[...]
    t_ref[...] = t
    s = jnp.concatenate([jnp.sum(t, axis=0, keepdims=True),
                         jnp.sum(t * t, axis=0, keepdims=True)], axis=0)
    prev = jnp.where(i == 0, jnp.zeros_like(s), s_ref[...])
    s_ref[...] = prev + s


def _k5(acc, dd, bias, width, nchunk):
    def body(*refs):
        _k5_body(nchunk, refs)
    in_specs = [pl.BlockSpec((1, NB, 128), functools.partial(
        lambda c, i: (c, i, 0), c)) for c in range(nchunk)]
    in_specs += [
        pl.BlockSpec((NB, 16), lambda i: (i, 0)),
        pl.BlockSpec((1, width), lambda i: (0, 0)),
    ]
    return pl.pallas_call(
        body,
        grid=(NBLK,),
        in_specs=in_specs,
        out_specs=[
            pl.BlockSpec((NB, width), lambda i: (i, 0)),
            pl.BlockSpec((2, width), lambda i: (0, 0)),
        ],
        out_shape=[
            jax.ShapeDtypeStruct((N, width), jnp.float32),
            jax.ShapeDtypeStruct((2, width), jnp.float32),
        ],
    )(*([acc] * nchunk), dd, bias)


def _k6_body(t_ref, s_ref, g_ref, be_ref, w3_ref, dd_ref, out_ref):
    s = s_ref[...]
    mu = s[0:1] / N
    var = s[1:2] / N - mu * mu
    h3 = (t_ref[...] - mu) * lax.rsqrt(var + 1e-5) * g_ref[...] + be_ref[...]
    h3 = jnp.maximum(h3, 0.0)
    dinv = _dinv_from_dd(dd_ref[...])
    out_ref[0] = dinv * jnp.dot(h3, w3_ref[...],
                                preferred_element_type=jnp.float32, precision=lax.Precision.HIGHEST)


def _k6(t, stats, g1, be1, w3, dd):
    return pl.pallas_call(
        _k6_body,
        grid=(NBLK, 8),
        in_specs=[
            pl.BlockSpec((NB, 512), lambda i, c: (i, 0)),
            pl.BlockSpec((2, 512), lambda i, c: (0, 0)),
            pl.BlockSpec((1, 512), lambda i, c: (0, 0)),
            pl.BlockSpec((1, 512), lambda i, c: (0, 0)),
            pl.BlockSpec((512, 128), lambda i, c: (0, c)),
            pl.BlockSpec((NB, 16), lambda i, c: (i, 0)),
        ],
        out_specs=pl.BlockSpec((1, NB, 128), lambda i, c: (c, i, 0)),
        out_shape=jax.ShapeDtypeStruct((8, N, 128), jnp.float32),
    )(t, stats, g1, be1, w3, dd)


def _k7b_body(s_ref, g_ref, be_ref, ab_ref):
    s = s_ref[...]
    mu = s[0:1] / N
    var = s[1:2] / N - mu * mu
    a = g_ref[...] * lax.rsqrt(var + 1e-5)
    b = be_ref[...] - mu * a
    ab_ref[...] = jnp.concatenate([a, b], axis=0)


def _k7b(stats, g2, be2):
    return pl.pallas_call(
        _k7b_body,
        grid=(1,),
        in_specs=[
            pl.BlockSpec((2, 1024), lambda i: (0, 0)),
            pl.BlockSpec((1, 1024), lambda i: (0, 0)),
            pl.BlockSpec((1, 1024), lambda i: (0, 0)),
        ],
        out_specs=pl.BlockSpec((2, 1024), lambda i: (0, 0)),
        out_shape=jax.ShapeDtypeStruct((2, 1024), jnp.float32),
    )(stats, g2, be2)


def _k8_body(p_ref, w1_ref, b1_ref, w2_ref, b2_ref, out_ref):
    z = jnp.dot(p_ref[...], w1_ref[...], preferred_element_type=jnp.float32, precision=lax.Precision.HIGHEST)
    z = jnp.maximum(z + b1_ref[...], 0.0)
    out_ref[...] = jnp.dot(z, w2_ref[...],
                           preferred_element_type=jnp.float32, precision=lax.Precision.HIGHEST) + b2_ref[...]


def _k8(pooled, w_fc1, b_fc1, w_fc2p, b_fc2p):
    return pl.pallas_call(
        _k8_body,
        grid=(1,),
        in_specs=[
            pl.BlockSpec((G, 2048), lambda i: (0, 0)),
            pl.BlockSpec((2048, 256), lambda i: (0, 0)),
            pl.BlockSpec((1, 256), lambda i: (0, 0)),
            pl.BlockSpec((256, 128), lambda i: (0, 0)),
            pl.BlockSpec((1, 128), lambda i: (0, 0)),
        ],
        out_specs=pl.BlockSpec((G, 128), lambda i: (0, 0)),
        out_shape=jax.ShapeDtypeStruct((G, 128), jnp.float32),
    )(pooled, w_fc1, b_fc1, w_fc2p, b_fc2p)


EPT = EP // 16
NBATCH = EPT // 128
RPT = N // 16


def _sc_mesh():
    return plsc.VectorSubcoreMesh(core_axis_name="c", subcore_axis_name="s",
                                  num_cores=2, num_subcores=16)


def _lane():
    return jnp.arange(16, dtype=jnp.int32)


def _zero_zbuf(zbuf, width):
    nj = width // 16

    def zb(i, c):
        for j in range(nj):
            zbuf[i, pl.ds(16 * j, 16)] = jnp.zeros((16,), jnp.float32)
        return c
    lax.fori_loop(0, zbuf.shape[0], zb, None)


def _zero_acc(zbuf, acc, sid):
    r0 = sid * RPT
    rows = zbuf.shape[0]
    for k in range(RPT // rows):
        pltpu.sync_copy(zbuf, acc.at[pl.ds(r0 + k * rows, rows)])


def _sc_b1_body(arows, arowd, srcp, dstp, wp, dd_out, exw_out,
                idx_s, idx_d, sbuf, dbuf, robuf, wbuf, zbuf, sem, acc):
    cid = lax.axis_index("c")
    sid = lax.axis_index("s")
    lmask = _lane() < 8

    @pl.when(cid == 0)
    def _():
        def zb(i, c):
            zbuf[i] = jnp.zeros((16,), jnp.float32)
            return c
        lax.fori_loop(0, 125, zb, None)
        _zero_acc(zbuf, acc, sid)
        plsc.subcore_barrier()
        ebase = sid * EPT

        def batch_body(b, c):
            base = ebase + b * 128
            pltpu.sync_copy(srcp.at[pl.ds(base, 128)], idx_s)
            pltpu.sync_copy(dstp.at[pl.ds(base, 128)], idx_d)
            pltpu.sync_copy(wp.at[pl.ds(base, 128)], wbuf)
            pltpu.async_copy(arows.at[idx_s], sbuf, sem).wait()
            pltpu.async_copy(arowd.at[idx_d], dbuf, sem).wait()

            def edge_body(i, c2):
                a = sbuf[i] + dbuf[i]
                a = jnp.maximum(a, 0.2 * a)
                ex = jnp.exp(a)
                valid = (base + i) < EV
                robuf[i] = jnp.where(jnp.logical_and(lmask, valid), ex, 0.0)
                return c2
            lax.fori_loop(0, 128, edge_body, None)
            for g in range(8):
                plsc.store_scatter(
                    robuf,
                    [_lane() + 16 * g, jnp.full((16,), 8, jnp.int32)],
                    wbuf[pl.ds(16 * g, 16)])
            pltpu.sync_copy(robuf, exw_out.at[pl.ds(base, 128)])
            pltpu.sync_copy(robuf, acc.at[idx_d], add=True)
            return c
        lax.fori_loop(0, NBATCH, batch_body, None)
        plsc.subcore_barrier()
        r0 = sid * RPT
        pltpu.sync_copy(acc.at[pl.ds(r0, RPT)], dd_out.at[pl.ds(r0, RPT)])


def _sc_b1(arows, arowd, srcp, dstp, wp):
    f = pl.kernel(
        _sc_b1_body,
        out_type=[jax.ShapeDtypeStruct((N, 16), jnp.float32),
                  jax.ShapeDtypeStruct((EP, 16), jnp.float32)],
        mesh=_sc_mesh(),
        compiler_params=pltpu.CompilerParams(use_tc_tiling_on_sc=False, needs_layout_passes=False),
        scratch_types=[
            pltpu.VMEM((128,), jnp.int32),
            pltpu.VMEM((128,), jnp.int32),
            pltpu.VMEM((128, 16), jnp.float32),
            pltpu.VMEM((128, 16), jnp.float32),
            pltpu.VMEM((128, 16), jnp.float32),
            pltpu.VMEM((128,), jnp.float32),
            pltpu.VMEM((125, 16), jnp.float32),
            pltpu.SemaphoreType.DMA,
            pltpu.VMEM_SHARED((N, 16), jnp.float32),
        ],
    )
    return f(arows, arowd, srcp, dstp, wp)


def _sc_b2_body(h1a, h1b, exw, dd, srcp, dstp, gata, gatb,
                idxsa, idxsb, idxda, idxdb, hbufa, hbufb, ddbufa, ddbufb,
                exbufa, exbufb, cfbuf, zbuf, gsema, gsemb, acc):
    cid = lax.axis_index("c")
    sid = lax.axis_index("s")
    idxs = (idxsa, idxsb)
    idxd = (idxda, idxdb)
    hbuf = (hbufa, hbufb)
    ddbuf = (ddbufa, ddbufb)
    exbuf = (exbufa, exbufb)
    gsem = (gsema, gsemb)

    def process(href, out_ref, headbase):
        _zero_zbuf(zbuf, 128)
        _zero_acc(zbuf, acc, sid)
        plsc.subcore_barrier()
        ebase = sid * EPT

        def prefetch(b, slot):
            base = ebase + b * 128
            pltpu.sync_copy(srcp.at[pl.ds(base, 128)], idxs[slot])
            pltpu.sync_copy(dstp.at[pl.ds(base, 128)], idxd[slot])
            pltpu.sync_copy(exw.at[pl.ds(base, 128)], exbuf[slot])
            pltpu.async_copy(href.at[idxs[slot]], hbuf[slot], gsem[slot])
            pltpu.async_copy(dd.at[idxd[slot]], ddbuf[slot], gsem[slot])

        def gwait(slot):
            pltpu.make_async_copy(
                href.at[idxs[slot]], hbuf[slot], gsem[slot]).wait()
            pltpu.make_async_copy(
                dd.at[idxd[slot]], ddbuf[slot], gsem[slot]).wait()

        prefetch(0, 0)

        def pair_body(k, c):
            for par in (0, 1):
                b = 2 * k + par
                nxt = 1 - par
                if par == 0:
                    prefetch(b + 1, nxt)
                else:
                    @pl.when(k < NBATCH // 2 - 1)
                    def _():
                        prefetch(b + 1, nxt)
                gwait(par)

                def edge_body(i, c2):
                    for u in range(2):
                        e = 2 * i + u
                        cfbuf[e] = exbuf[par][e] / ddbuf[par][e]
                        iv = jnp.zeros((16,), jnp.int32) + e
                        for hd in range(4):
                            cs = plsc.load_gather(
                                cfbuf,
                                [iv, jnp.full((16,), headbase + hd,
                                              jnp.int32)])
                            for j in (2 * hd, 2 * hd + 1):
                                hbuf[par][e, pl.ds(16 * j, 16)] = (
                                    hbuf[par][e, pl.ds(16 * j, 16)] * cs)
                    return c2
                lax.fori_loop(0, 64, edge_body, None)
                pltpu.sync_copy(hbuf[par], acc.at[idxd[par]], add=True)
            return c
        lax.fori_loop(0, NBATCH // 2, pair_body, None)
        plsc.subcore_barrier()
        r0 = sid * RPT
        pltpu.sync_copy(acc.at[pl.ds(r0, RPT)], out_ref.at[pl.ds(r0, RPT)])

    pl.when(cid == 0)(lambda: process(h1a, gata, 0))
    pl.when(cid == 1)(lambda: process(h1b, gatb, 4))


def _sc_b2(h1a, h1b, exw, dd, srcp, dstp):
    f = pl.kernel(
        _sc_b2_body,
        out_type=[jax.ShapeDtypeStruct((N, 128), jnp.float32),
                  jax.ShapeDtypeStruct((N, 128), jnp.float32)],
        mesh=_sc_mesh(),
        compiler_params=pltpu.CompilerParams(use_tc_tiling_on_sc=False, needs_layout_passes=False),
        scratch_types=[
            pltpu.VMEM((128,), jnp.int32),
            pltpu.VMEM((128,), jnp.int32),
            pltpu.VMEM((128,), jnp.int32),
            pltpu.VMEM((128,), jnp.int32),
            pltpu.VMEM((128, 128), jnp.float32),
            pltpu.VMEM((128, 128), jnp.float32),
            pltpu.VMEM((128, 16), jnp.float32),
            pltpu.VMEM((128, 16), jnp.float32),
            pltpu.VMEM((128, 16), jnp.float32),
            pltpu.VMEM((128, 16), jnp.float32),
            pltpu.VMEM((128, 16), jnp.float32),
            pltpu.VMEM((25, 128), jnp.float32),
            pltpu.SemaphoreType.DMA,
            pltpu.SemaphoreType.DMA,
            pltpu.VMEM_SHARED((N, 128), jnp.float32),
        ],
    )
    return f(h1a, h1b, exw, dd, srcp, dstp)


def _sc_gcn_body(nc_per_sc, hs_flat, srcp, dstp, wp, out_flat,
                 idx_s, idx2a, idx2b, idxda, idxdb, hbufa, hbufb,
                 wbufa, wbufb, zbuf, gsema, gsemb, acc):
    cid = lax.axis_index("c")
    sid = lax.axis_index("s")
    idx2 = (idx2a, idx2b)
    idxd = (idxda, idxdb)
    hbuf = (hbufa, hbufb)
    wbuf = (wbufa, wbufb)
    gsem = (gsema, gsemb)
    _zero_zbuf(zbuf, 128)
    ebase = sid * EPT
    r0 = sid * RPT
    for cl in range(nc_per_sc):
        cg = cid * nc_per_sc + cl
        off = cg * N
        _zero_acc(zbuf, acc, sid)
        plsc.subcore_barrier()

        def prefetch(b, slot):
            base = ebase + b * 128
            pltpu.sync_copy(srcp.at[pl.ds(base, 128)], idx_s)
            pltpu.sync_copy(dstp.at[pl.ds(base, 128)], idxd[slot])
            pltpu.sync_copy(wp.at[pl.ds(base, 128)], wbuf[slot])
            for g in range(8):
                idx2[slot][pl.ds(16 * g, 16)] = (
                    idx_s[pl.ds(16 * g, 16)] + off)
            pltpu.async_copy(hs_flat.at[idx2[slot]], hbuf[slot], gsem[slot])

        def gwait(slot):
            pltpu.make_async_copy(
                hs_flat.at[idx2[slot]], hbuf[slot], gsem[slot]).wait()

        prefetch(0, 0)

        def pair_body(k, c):
            for par in (0, 1):
                b = 2 * k + par
                nxt = 1 - par
                if par == 0:
                    prefetch(b + 1, nxt)
                else:
                    @pl.when(k < NBATCH // 2 - 1)
                    def _():
                        prefetch(b + 1, nxt)
                gwait(par)

                def edge_body(i, c2):
                    for u in range(2):
                        e = 2 * i + u
                        iv = jnp.zeros((16,), jnp.int32) + e
                        ws = plsc.load_gather(wbuf[par], [iv])
                        for j in range(8):
                            hbuf[par][e, pl.ds(16 * j, 16)] = (
                                hbuf[par][e, pl.ds(16 * j, 16)] * ws)
                    return c2
                lax.fori_loop(0, 64, edge_body, None)
                pltpu.sync_copy(hbuf[par], acc.at[idxd[par]], add=True)
            return c
        lax.fori_loop(0, NBATCH // 2, pair_body, None)
        plsc.subcore_barrier()
        pltpu.sync_copy(acc.at[pl.ds(r0, RPT)],
                        out_flat.at[pl.ds(off + r0, RPT)])
        plsc.subcore_barrier()


def _sc_gcn(hchunks, srcp, dstp, wp):
    nc = hchunks.shape[0]
    hs_flat = hchunks.reshape(nc * N, 128)
    f = pl.kernel(
        functools.partial(_sc_gcn_body, nc // 2),
        out_type=jax.ShapeDtypeStruct((nc * N, 128), jnp.float32),
        mesh=_sc_mesh(),
        compiler_params=pltpu.CompilerParams(use_tc_tiling_on_sc=False, needs_layout_passes=False),
        scratch_types=[
            pltpu.VMEM((128,), jnp.int32),
            pltpu.VMEM((128,), jnp.int32),
            pltpu.VMEM((128,), jnp.int32),
            pltpu.VMEM((128,), jnp.int32),
            pltpu.VMEM((128,), jnp.int32),
            pltpu.VMEM((128, 128), jnp.float32),
            pltpu.VMEM((128, 128), jnp.float32),
            pltpu.VMEM((128,), jnp.float32),
            pltpu.VMEM((128,), jnp.float32),
            pltpu.VMEM((25, 128), jnp.float32),
            pltpu.SemaphoreType.DMA,
            pltpu.SemaphoreType.DMA,
            pltpu.VMEM_SHARED((N, 128), jnp.float32),
        ],
    )
    return f(hs_flat, srcp, dstp, wp).reshape(nc, N, 128)


def _sc_pool_body(u, ab, rp, pooled,
                  ubuf, abbuf, obmax, obmean, rp_s, sem):
    cid = lax.axis_index("c")
    sid = lax.axis_index("s")
    wid = cid * 16 + sid
    c0 = wid * 32
    pltpu.sync_copy(rp, rp_s)
    pltpu.sync_copy(ab.at[:, pl.ds(c0, 32)], abbuf)
    a0 = abbuf[0, pl.ds(0, 16)]
    a1 = abbuf[0, pl.ds(16, 16)]
    b0 = abbuf[1, pl.ds(0, 16)]
    b1 = abbuf[1, pl.ds(16, 16)]
    rpv0 = rp_s[pl.ds(0, 16)]
    rpv1 = rp_s[pl.ds(16, 16)]
    for g in range(G):
        lo = rpv0[g]
        hi = rpv0[g + 1] if g + 1 < 16 else rpv1[0]
        cnt = hi - lo
        nblk = lax.div(cnt + 511, 512)

        def blk(bi, carry):
            blo = lo + bi * 512
            s = jnp.minimum(blo, N - 512)
            pltpu.sync_copy(u.at[pl.ds(s, 512), pl.ds(c0, 32)], ubuf)

            def row(i, c2):
                m0, m1, s0, s1 = c2
                r = s + i
                valid = jnp.logical_and(r >= blo, r < hi)
                v0 = jnp.maximum(ubuf[i, pl.ds(0, 16)] * a0 + b0, 0.0)
                v1 = jnp.maximum(ubuf[i, pl.ds(16, 16)] * a1 + b1, 0.0)
                v0 = jnp.where(valid, v0, 0.0)
                v1 = jnp.where(valid, v1, 0.0)
                return (jnp.maximum(m0, v0), jnp.maximum(m1, v1),
                        s0 + v0, s1 + v1)
            return lax.fori_loop(0, 512, row, carry)
        z = jnp.zeros((16,), jnp.float32)
        m0, m1, s0, s1 = lax.fori_loop(0, nblk, blk, (z, z, z, z))
        cntf = jnp.maximum(cnt.astype(jnp.float32), 1.0)
        obmax[g, pl.ds(0, 16)] = m0
        obmax[g, pl.ds(16, 16)] = m1
        obmean[g, pl.ds(0, 16)] = s0 / cntf
        obmean[g, pl.ds(16, 16)] = s1 / cntf
    pltpu.sync_copy(obmax, pooled.at[:, pl.ds(c0, 32)])
    pltpu.sync_copy(obmean, pooled.at[:, pl.ds(1024 + c0, 32)])


def _sc_pool(u, ab, rp32):
    f = pl.kernel(
        _sc_pool_body,
        out_type=jax.ShapeDtypeStruct((G, 2048), jnp.float32),
        mesh=_sc_mesh(),
        compiler_params=pltpu.CompilerParams(use_tc_tiling_on_sc=False, needs_layout_passes=False),
        scratch_types=[
            pltpu.VMEM((512, 32), jnp.float32),
            pltpu.VMEM((2, 32), jnp.float32),
            pltpu.VMEM((G, 32), jnp.float32),
            pltpu.VMEM((G, 32), jnp.float32),
            pltpu.VMEM((32,), jnp.int32),
            pltpu.SemaphoreType.DMA,
        ],
    )
    return f(u, ab, rp32)


def kernel(x, edge_index, edge_weights, batch, W_gat, att_src, att_dst,
           b_gat, W2, b2, W3, b3, g1, be1, g2, be2, W_fc1, b_fc1,
           W_fc2, b_fc2):
    loop = jnp.arange(N, dtype=edge_index.dtype)
    pad = jnp.zeros((EP - EV,), jnp.int32)
    srcp = jnp.concatenate([edge_index[0], loop, pad])
    dstp = jnp.concatenate([edge_index[1], loop, pad])
    wp = jnp.concatenate([edge_weights, jnp.ones((N,), jnp.float32),
                          jnp.zeros((EP - EV,), jnp.float32)])
    eye = jnp.eye(H, dtype=jnp.float32)
    zpad = jnp.zeros((H * C, 8), jnp.float32)
    asads = jnp.concatenate(
        [(eye[:, None, :] * att_src[:, :, None]).reshape(H * C, H), zpad], 1)
    asadd = jnp.concatenate(
        [(eye[:, None, :] * att_dst[:, :, None]).reshape(H * C, H), zpad], 1)
    rp = jnp.searchsorted(batch, jnp.arange(G + 1, dtype=jnp.int32)
                          ).astype(jnp.int32)
    rp32 = jnp.concatenate([rp, jnp.full((15,), N, jnp.int32)])
    w_fc2p = jnp.concatenate([W_fc2, jnp.zeros((256, 126), jnp.float32)], 1)
    b_fc2p = jnp.concatenate([b_fc2, jnp.zeros((126,), jnp.float32)])

    h1a, h1b, arows, arowd = _k1(x, W_gat, asads, asadd)
    dd, exw = _sc_b1(arows, arowd, srcp, dstp, wp)
    gata, gatb = _sc_b2(h1a, h1b, exw, dd, srcp, dstp)
    h2s = _k3(gata, gatb, dd, b_gat.reshape(1, -1), W2)
    g1acc = _sc_gcn(h2s, srcp, dstp, wp)
    t, stats1 = _k5(g1acc, dd, b2.reshape(1, -1), 512, 4)
    h3s = _k6(t, stats1, g1.reshape(1, -1), be1.reshape(1, -1), W3, dd)
    g2acc = _sc_gcn(h3s, srcp, dstp, wp)
    u, stats2 = _k5(g2acc, dd, b3.reshape(1, -1), 1024, 8)
    ab = _k7b(stats2, g2.reshape(1, -1), be2.reshape(1, -1))
    pooled = _sc_pool(u, ab, rp32)
    z = _k8(pooled, W_fc1, b_fc1.reshape(1, -1), w_fc2p,
            b_fc2p.reshape(1, -1))
    return z[:, :2]
